# baseline (speedup 1.0000x reference)
import functools

import jax
import jax.numpy as jnp
from jax import lax
from jax.experimental import pallas as pl
from jax.experimental.pallas import tpu as pltpu
from jax.experimental.pallas import tpu_sc as plsc

F32 = jnp.float32
CHUNK = 128
NW = 32


def _proj_body(nf_ref, ws_ref, wr_ref, ps_ref, pr_ref):
    nf = nf_ref[...]
    ps_ref[...] = jnp.dot(nf, ws_ref[...], preferred_element_type=F32)
    pr_ref[...] = jnp.dot(nf, wr_ref[...], preferred_element_type=F32)


def _edge_body(ef_ref, gs_ref, gr_ref, we_ref, be1_ref, we2_ref, be2_ref,
               sc_ref, bi_ref, ue_ref, ne_ref):
    ef = ef_ref[...]
    x = (gs_ref[0] + gr_ref[0] + be1_ref[...]
         + jnp.dot(ef, we_ref[...], preferred_element_type=F32))
    h = jnp.maximum(x, 0.0)
    y = jnp.dot(h, we2_ref[...], preferred_element_type=F32) + be2_ref[...]
    mean = jnp.mean(y, axis=-1, keepdims=True)
    var = jnp.mean((y - mean) ** 2, axis=-1, keepdims=True)
    ue = (y - mean) / jnp.sqrt(var + 1e-5) * sc_ref[...] + bi_ref[...]
    ue_ref[...] = ue
    ne_ref[...] = ue + ef


def _node_body(nf_ref, p0_ref, p1_ref, p2_ref, p3_ref, wa_ref, wb_ref,
               bn1_ref, wn2_ref, bn2_ref, sc_ref, bi_ref, out_ref):
    nf = nf_ref[...]
    agg = (p0_ref[0] + p1_ref[0]) + (p2_ref[0] + p3_ref[0])
    x = (jnp.dot(nf, wa_ref[...], preferred_element_type=F32)
         + jnp.dot(agg, wb_ref[...], preferred_element_type=F32)
         + bn1_ref[...])
    h = jnp.maximum(x, 0.0)
    y = jnp.dot(h, wn2_ref[...], preferred_element_type=F32) + bn2_ref[...]
    mean = jnp.mean(y, axis=-1, keepdims=True)
    var = jnp.mean((y - mean) ** 2, axis=-1, keepdims=True)
    out_ref[...] = ((y - mean) / jnp.sqrt(var + 1e-5) * sc_ref[...]
                    + bi_ref[...] + nf)


def _tc_proj(nf, ws, wr):
    n, d = nf.shape
    bn = 2000
    return pl.pallas_call(
        _proj_body,
        grid=(n // bn,),
        in_specs=[
            pl.BlockSpec((bn, d), lambda i: (i, 0)),
            pl.BlockSpec((d, d), lambda i: (0, 0)),
            pl.BlockSpec((d, d), lambda i: (0, 0)),
        ],
        out_specs=(pl.BlockSpec((bn, d), lambda i: (i, 0)),
                   pl.BlockSpec((bn, d), lambda i: (i, 0))),
        out_shape=(jax.ShapeDtypeStruct((n, d), F32),
                   jax.ShapeDtypeStruct((n, d), F32)),
    )(nf, ws, wr)


def _tc_edge(ef, g2, we, be1, we2, be2, sc, bi, row_off, nrows):
    d = ef.shape[1]
    be = 1280
    assert row_off % be == 0 and nrows % be == 0
    ob = row_off // be
    row = lambda i: (i, 0)
    cst = lambda i: (0, 0)
    return pl.pallas_call(
        _edge_body,
        grid=(nrows // be,),
        in_specs=[
            pl.BlockSpec((be, d), lambda i: (ob + i, 0)),
            pl.BlockSpec((1, be, d), lambda i: (0, i, 0)),
            pl.BlockSpec((1, be, d), lambda i: (1, i, 0)),
            pl.BlockSpec((d, d), cst),
            pl.BlockSpec((1, d), cst),
            pl.BlockSpec((d, d), cst),
            pl.BlockSpec((1, d), cst),
            pl.BlockSpec((1, d), cst),
            pl.BlockSpec((1, d), cst),
        ],
        out_specs=(pl.BlockSpec((be, d), row), pl.BlockSpec((be, d), row)),
        out_shape=(jax.ShapeDtypeStruct((nrows, d), F32),
                   jax.ShapeDtypeStruct((nrows, d), F32)),
    )(ef, g2, g2, we, be1, we2, be2, sc, bi)


def _tc_node(nf, agg_a, agg_b, wa, wb, bn1, wn2, bn2, sc, bi):
    n, d = nf.shape
    bn = 2000
    row = lambda i: (i, 0)
    cst = lambda i: (0, 0)
    p_a = lambda k: pl.BlockSpec((1, bn, d), lambda i, _k=k: (_k, i, 0))
    return pl.pallas_call(
        _node_body,
        grid=(n // bn,),
        in_specs=[
            pl.BlockSpec((bn, d), row),
            p_a(0),
            p_a(1),
            p_a(0),
            p_a(1),
            pl.BlockSpec((d, d), cst),
            pl.BlockSpec((d, d), cst),
            pl.BlockSpec((1, d), cst),
            pl.BlockSpec((d, d), cst),
            pl.BlockSpec((1, d), cst),
            pl.BlockSpec((1, d), cst),
            pl.BlockSpec((1, d), cst),
        ],
        out_specs=pl.BlockSpec((bn, d), row),
        out_shape=jax.ShapeDtypeStruct((n, d), F32),
    )(nf, agg_a, agg_a, agg_b, agg_b, wa, wb, bn1, wn2, bn2, sc, bi)


@functools.lru_cache(maxsize=None)
def _make_gather(n_nodes, n_chunks, chunk_off, d):
    mesh = plsc.VectorSubcoreMesh(core_axis_name="c", subcore_axis_name="s")
    e_out = n_chunks * CHUNK
    NS = 16
    n_full = n_chunks // NS
    extra = n_chunks % NS
    assert n_full >= 4 and n_full % 2 == 0
    stripe = ((n_nodes // NS) // 8 + 1) * 8
    last = n_nodes - 15 * stripe
    assert 0 < last <= stripe

    @functools.partial(
        pl.kernel,
        out_type=jax.ShapeDtypeStruct((2, e_out, d), F32),
        mesh=mesh,
        scratch_types=[
            pltpu.VMEM((CHUNK,), jnp.int32),
            pltpu.VMEM((CHUNK,), jnp.int32),
            pltpu.VMEM((CHUNK, d), F32),
            pltpu.VMEM((CHUNK, d), F32),
            pltpu.VMEM_SHARED((n_nodes, d), F32),
            pltpu.SemaphoreType.DMA,
            pltpu.SemaphoreType.DMA,
        ],
    )
    def gather_k(ps_hbm, pr_hbm, sidx_hbm, ridx_hbm, out_hbm,
                 ib0, ib1, rv0, rv1, tbl_sh, ws0, ws1):
        cid = lax.axis_index("c")
        sid = lax.axis_index("s")
        B = ((ib0, rv0, ws0), (ib1, rv1, ws1))

        @pl.when(sid < 15)
        def _():
            sl = pl.ds(sid * stripe, stripe)

            @pl.when(cid == 0)
            def _():
                pltpu.sync_copy(ps_hbm.at[sl], tbl_sh.at[sl])

            @pl.when(cid == 1)
            def _():
                pltpu.sync_copy(pr_hbm.at[sl], tbl_sh.at[sl])

        @pl.when(sid == 15)
        def _():
            sl = pl.ds(15 * stripe, last)

            @pl.when(cid == 0)
            def _():
                pltpu.sync_copy(ps_hbm.at[sl], tbl_sh.at[sl])

            @pl.when(cid == 1)
            def _():
                pltpu.sync_copy(pr_hbm.at[sl], tbl_sh.at[sl])

        plsc.subcore_barrier()

        def load_idx(j, b):
            ib = B[b][0]
            cg = chunk_off + j * NS + sid

            @pl.when(cid == 0)
            def _():
                pltpu.sync_copy(sidx_hbm.at[pl.ds(cg * CHUNK, CHUNK)], ib)

            @pl.when(cid == 1)
            def _():
                pltpu.sync_copy(ridx_hbm.at[pl.ds(cg * CHUNK, CHUNK)], ib)

        def stage(j, b, first):
            ib, rv, wsem = B[b]
            c = j * NS + sid
            if not first:
                pltpu.make_async_copy(
                    rv, out_hbm.at[cid, pl.ds(0, CHUNK)], wsem).wait()
            load_idx(j + 1, 1 - b)
            pltpu.sync_copy(tbl_sh.at[ib], rv)
            pltpu.async_copy(rv, out_hbm.at[cid, pl.ds(c * CHUNK, CHUNK)],
                             wsem)

        load_idx(0, 0)
        stage(0, 0, True)
        stage(1, 1, True)

        def body(i, carry):
            stage(2 * i, 0, False)
            stage(2 * i + 1, 1, False)
            return carry

        lax.fori_loop(1, n_full // 2, body, 0)
        if extra:
            @pl.when(sid < extra)
            def _():
                stage(n_full, 0, False)
        pltpu.make_async_copy(rv0, out_hbm.at[cid, pl.ds(0, CHUNK)],
                              ws0).wait()
        pltpu.make_async_copy(rv1, out_hbm.at[cid, pl.ds(0, CHUNK)],
                              ws1).wait()

    return gather_k


@functools.lru_cache(maxsize=None)
def _make_scatter(n_nodes, n_chunks, chunk_off, d):
    mesh = plsc.VectorSubcoreMesh(core_axis_name="c", subcore_axis_name="s")
    n_pad = ((n_nodes + 16 * 128 - 1) // (16 * 128)) * 16 * 128
    rows_per_tile = n_pad // 16
    zr = 64
    n_zcopy = rows_per_tile // zr

    n_full = n_chunks // NW
    extra = n_chunks % NW
    assert n_full >= 4 and n_full % 2 == 0

    @functools.partial(
        pl.kernel,
        out_type=jax.ShapeDtypeStruct((2, n_pad, d), F32),
        mesh=mesh,
        scratch_types=[
            pltpu.VMEM((CHUNK,), jnp.int32),
            pltpu.VMEM((CHUNK,), jnp.int32),
            pltpu.VMEM((CHUNK, d), F32),
            pltpu.VMEM((CHUNK, d), F32),
            pltpu.VMEM((zr, d), F32),
            pltpu.VMEM_SHARED((n_pad, d), F32),
            pltpu.SemaphoreType.DMA,
            pltpu.SemaphoreType.DMA,
        ],
    )
    def scatter_k(ue_hbm, ridx_hbm, out_hbm, ib0, ib1, rv0, rv1, zbuf,
                  acc_sh, sm0, sm1):
        cid = lax.axis_index("c")
        sid = lax.axis_index("s")
        wid = sid * 2 + cid
        B = ((ib0, rv0, sm0), (ib1, rv1, sm1))

        def issue(j, b):
            ib, rv, sm = B[b]
            c = j * NW + wid
            cg = chunk_off + c
            pltpu.sync_copy(ridx_hbm.at[pl.ds(cg * CHUNK, CHUNK)], ib)
            pltpu.async_copy(ue_hbm.at[pl.ds(c * CHUNK, CHUNK)], rv, sm)

        def finish(j, b):
            ib, rv, sm = B[b]
            c = j * NW + wid
            pltpu.make_async_copy(
                ue_hbm.at[pl.ds(c * CHUNK, CHUNK)], rv, sm).wait()
            pltpu.sync_copy(rv, acc_sh.at[ib], add=True)

        issue(0, 0)

        def zb(i, carry):
            r = i // (d // 16)
            q = (i % (d // 16)) * 16
            zbuf[r, pl.ds(q, 16)] = jnp.zeros((16,), F32)
            return carry

        lax.fori_loop(0, zr * (d // 16), zb, 0)
        base = sid * rows_per_tile
        for t in range(n_zcopy):
            pltpu.sync_copy(zbuf, acc_sh.at[pl.ds(base + t * zr, zr)])
        plsc.subcore_barrier()

        def body(i, carry):
            j0 = 2 * i
            issue(j0 + 1, 1)
            finish(j0, 0)
            issue(j0 + 2, 0)
            finish(j0 + 1, 1)
            return carry

        lax.fori_loop(0, n_full // 2 - 1, body, 0)
        issue(n_full - 1, 1)
        finish(n_full - 2, 0)
        if extra:
            @pl.when(wid < extra)
            def _():
                issue(n_full, 0)
        finish(n_full - 1, 1)
        if extra:
            @pl.when(wid < extra)
            def _():
                finish(n_full, 0)
        plsc.subcore_barrier()
        for t in range(n_zcopy):
            sl = pl.ds(base + t * zr, zr)
            pltpu.sync_copy(acc_sh.at[sl], out_hbm.at[cid, sl])

    return scatter_k


def kernel(node_features, edge_features, senders, receivers,
           We1, be1, We2, be2, ln_e_scale, ln_e_bias,
           Wn1, bn1, Wn2, bn2, ln_n_scale, ln_n_bias):
    n, d = node_features.shape
    e = edge_features.shape[0]
    n_chunks = e // CHUNK

    sidx = senders.astype(jnp.int32)
    ridx = receivers.astype(jnp.int32)

    n_chunks_pad = ((n_chunks + 2 * NW - 1) // (2 * NW)) * 2 * NW
    e_pad = n_chunks_pad * CHUNK
    sidx_p = jnp.pad(sidx, (0, e_pad - e))
    ridx_p = jnp.pad(ridx, (0, e_pad - e))

    c0 = 1280
    assert n_chunks > c0
    phases = ((0, c0), (c0, n_chunks - c0))

    ws, wr, we = We1[:d], We1[d:2 * d], We1[2 * d:]
    ps, pr = _tc_proj(node_features, ws, wr)

    be1r, be2r = be1.reshape(1, d), be2.reshape(1, d)
    scr, bir = ln_e_scale.reshape(1, d), ln_e_bias.reshape(1, d)
    ue_p, ne_p, agg_p = [], [], []
    for off, nc in phases:
        g2 = _make_gather(n, nc, off, d)(ps, pr, sidx_p, ridx_p)
        ue, ne = _tc_edge(edge_features, g2, we, be1r, We2, be2r, scr, bir,
                          off * CHUNK, nc * CHUNK)
        agg = _make_scatter(n, nc, off, d)(ue, ridx_p)
        ue_p.append(ue)
        ne_p.append(ne)
        agg_p.append(agg)

    new_edges = jnp.concatenate(ne_p, axis=0)
    new_nodes = _tc_node(node_features, agg_p[0][:, :n], agg_p[1][:, :n],
                         Wn1[:d], Wn1[d:], bn1.reshape(1, d),
                         Wn2, bn2.reshape(1, d),
                         ln_n_scale.reshape(1, d), ln_n_bias.reshape(1, d))
    return (new_nodes, new_edges)


# speedup vs baseline: 1.0567x; 1.0567x over previous
import functools

import jax
import jax.numpy as jnp
from jax import lax
from jax.experimental import pallas as pl
from jax.experimental.pallas import tpu as pltpu
from jax.experimental.pallas import tpu_sc as plsc

F32 = jnp.float32
CHUNK = 128
NW = 32


def _proj_body(nf_ref, ws_ref, wr_ref, ps_ref, pr_ref):
    nf = nf_ref[...]
    ps_ref[...] = jnp.dot(nf, ws_ref[...], preferred_element_type=F32)
    pr_ref[...] = jnp.dot(nf, wr_ref[...], preferred_element_type=F32)


def _edge_body(ef_ref, gs_ref, gr_ref, we_ref, be1_ref, we2_ref, be2_ref,
               sc_ref, bi_ref, ue_ref, ne_ref):
    ef = ef_ref[...]
    x = (gs_ref[0] + gr_ref[0] + be1_ref[...]
         + jnp.dot(ef, we_ref[...], preferred_element_type=F32))
    h = jnp.maximum(x, 0.0)
    y = jnp.dot(h, we2_ref[...], preferred_element_type=F32) + be2_ref[...]
    mean = jnp.mean(y, axis=-1, keepdims=True)
    var = jnp.mean((y - mean) ** 2, axis=-1, keepdims=True)
    ue = (y - mean) / jnp.sqrt(var + 1e-5) * sc_ref[...] + bi_ref[...]
    ue_ref[...] = ue
    ne_ref[...] = ue + ef


def _node_body(nf_ref, p0_ref, p1_ref, wa_ref, wb_ref, bn1_ref, wn2_ref,
               bn2_ref, sc_ref, bi_ref, out_ref):
    nf = nf_ref[...]
    agg = p0_ref[...] + p1_ref[...]
    x = (jnp.dot(nf, wa_ref[...], preferred_element_type=F32)
         + jnp.dot(agg, wb_ref[...], preferred_element_type=F32)
         + bn1_ref[...])
    h = jnp.maximum(x, 0.0)
    y = jnp.dot(h, wn2_ref[...], preferred_element_type=F32) + bn2_ref[...]
    mean = jnp.mean(y, axis=-1, keepdims=True)
    var = jnp.mean((y - mean) ** 2, axis=-1, keepdims=True)
    out_ref[...] = ((y - mean) / jnp.sqrt(var + 1e-5) * sc_ref[...]
                    + bi_ref[...] + nf)


def _tc_proj(nf, ws, wr):
    n, d = nf.shape
    bn = 2000
    return pl.pallas_call(
        _proj_body,
        grid=(n // bn,),
        in_specs=[
            pl.BlockSpec((bn, d), lambda i: (i, 0)),
            pl.BlockSpec((d, d), lambda i: (0, 0)),
            pl.BlockSpec((d, d), lambda i: (0, 0)),
        ],
        out_specs=(pl.BlockSpec((bn, d), lambda i: (i, 0)),
                   pl.BlockSpec((bn, d), lambda i: (i, 0))),
        out_shape=(jax.ShapeDtypeStruct((n, d), F32),
                   jax.ShapeDtypeStruct((n, d), F32)),
    )(nf, ws, wr)


def _tc_edge(ef, g2, we, be1, we2, be2, sc, bi):
    e, d = ef.shape
    be = 2000
    row = lambda i: (i, 0)
    cst = lambda i: (0, 0)
    return pl.pallas_call(
        _edge_body,
        grid=(e // be,),
        in_specs=[
            pl.BlockSpec((be, d), row),
            pl.BlockSpec((1, be, d), lambda i: (0, i, 0)),
            pl.BlockSpec((1, be, d), lambda i: (1, i, 0)),
            pl.BlockSpec((d, d), cst),
            pl.BlockSpec((1, d), cst),
            pl.BlockSpec((d, d), cst),
            pl.BlockSpec((1, d), cst),
            pl.BlockSpec((1, d), cst),
            pl.BlockSpec((1, d), cst),
        ],
        out_specs=(pl.BlockSpec((be, d), row), pl.BlockSpec((be, d), row)),
        out_shape=(jax.ShapeDtypeStruct((e, d), F32),
                   jax.ShapeDtypeStruct((e, d), F32)),
    )(ef, g2, g2, we, be1, we2, be2, sc, bi)


def _tc_node(nf, p0, p1, wa, wb, bn1, wn2, bn2, sc, bi):
    n, d = nf.shape
    bn = 2000
    row = lambda i: (i, 0)
    cst = lambda i: (0, 0)
    return pl.pallas_call(
        _node_body,
        grid=(n // bn,),
        in_specs=[
            pl.BlockSpec((bn, d), row),
            pl.BlockSpec((bn, d), row),
            pl.BlockSpec((bn, d), row),
            pl.BlockSpec((d, d), cst),
            pl.BlockSpec((d, d), cst),
            pl.BlockSpec((1, d), cst),
            pl.BlockSpec((d, d), cst),
            pl.BlockSpec((1, d), cst),
            pl.BlockSpec((1, d), cst),
            pl.BlockSpec((1, d), cst),
        ],
        out_specs=pl.BlockSpec((bn, d), row),
        out_shape=jax.ShapeDtypeStruct((n, d), F32),
    )(nf, p0, p1, wa, wb, bn1, wn2, bn2, sc, bi)


@functools.lru_cache(maxsize=None)
def _make_gather(n_nodes, n_chunks, n_chunks_pad, d):
    mesh = plsc.VectorSubcoreMesh(core_axis_name="c", subcore_axis_name="s")
    e_pad = n_chunks_pad * CHUNK
    NS = 16
    n_full = n_chunks // NS
    extra = n_chunks % NS
    assert n_full >= 4 and n_full % 2 == 0
    stripe = ((n_nodes // NS) // 8 + 1) * 8
    last = n_nodes - 15 * stripe
    assert 0 < last <= stripe

    @functools.partial(
        pl.kernel,
        out_type=jax.ShapeDtypeStruct((2, e_pad, d), F32),
        mesh=mesh,
        scratch_types=[
            pltpu.VMEM((CHUNK,), jnp.int32),
            pltpu.VMEM((CHUNK,), jnp.int32),
            pltpu.VMEM((CHUNK, d), F32),
            pltpu.VMEM((CHUNK, d), F32),
            pltpu.VMEM_SHARED((n_nodes, d), F32),
            pltpu.SemaphoreType.DMA,
            pltpu.SemaphoreType.DMA,
        ],
    )
    def gather_k(ps_hbm, pr_hbm, sidx_hbm, ridx_hbm, out_hbm,
                 ib0, ib1, rv0, rv1, tbl_sh, ws0, ws1):
        cid = lax.axis_index("c")
        sid = lax.axis_index("s")
        B = ((ib0, rv0, ws0), (ib1, rv1, ws1))

        @pl.when(sid < 15)
        def _():
            sl = pl.ds(sid * stripe, stripe)

            @pl.when(cid == 0)
            def _():
                pltpu.sync_copy(ps_hbm.at[sl], tbl_sh.at[sl])

            @pl.when(cid == 1)
            def _():
                pltpu.sync_copy(pr_hbm.at[sl], tbl_sh.at[sl])

        @pl.when(sid == 15)
        def _():
            sl = pl.ds(15 * stripe, last)

            @pl.when(cid == 0)
            def _():
                pltpu.sync_copy(ps_hbm.at[sl], tbl_sh.at[sl])

            @pl.when(cid == 1)
            def _():
                pltpu.sync_copy(pr_hbm.at[sl], tbl_sh.at[sl])

        plsc.subcore_barrier()

        def load_idx(j, b):
            ib = B[b][0]
            c = j * NS + sid

            @pl.when(cid == 0)
            def _():
                pltpu.sync_copy(sidx_hbm.at[pl.ds(c * CHUNK, CHUNK)], ib)

            @pl.when(cid == 1)
            def _():
                pltpu.sync_copy(ridx_hbm.at[pl.ds(c * CHUNK, CHUNK)], ib)

        def stage(j, b, first):
            ib, rv, wsem = B[b]
            c = j * NS + sid
            if not first:
                pltpu.make_async_copy(
                    rv, out_hbm.at[cid, pl.ds(0, CHUNK)], wsem).wait()
            load_idx(j + 1, 1 - b)
            pltpu.sync_copy(tbl_sh.at[ib], rv)
            pltpu.async_copy(rv, out_hbm.at[cid, pl.ds(c * CHUNK, CHUNK)],
                             wsem)

        load_idx(0, 0)
        stage(0, 0, True)
        stage(1, 1, True)

        def body(i, carry):
            stage(2 * i, 0, False)
            stage(2 * i + 1, 1, False)
            return carry

        lax.fori_loop(1, n_full // 2, body, 0)
        if extra:
            @pl.when(sid < extra)
            def _():
                stage(n_full, 0, False)
        pltpu.make_async_copy(rv0, out_hbm.at[cid, pl.ds(0, CHUNK)],
                              ws0).wait()
        pltpu.make_async_copy(rv1, out_hbm.at[cid, pl.ds(0, CHUNK)],
                              ws1).wait()

    return gather_k


@functools.lru_cache(maxsize=None)
def _make_scatter(n_nodes, n_chunks, d):
    mesh = plsc.VectorSubcoreMesh(core_axis_name="c", subcore_axis_name="s")
    n_pad = ((n_nodes + 16 * 128 - 1) // (16 * 128)) * 16 * 128
    rows_per_tile = n_pad // 16
    zr = 64
    n_zcopy = rows_per_tile // zr

    n_full = n_chunks // NW
    extra = n_chunks % NW
    assert n_full >= 4 and n_full % 2 == 0

    @functools.partial(
        pl.kernel,
        out_type=jax.ShapeDtypeStruct((2, n_pad, d), F32),
        mesh=mesh,
        scratch_types=[
            pltpu.VMEM((CHUNK,), jnp.int32),
            pltpu.VMEM((CHUNK,), jnp.int32),
            pltpu.VMEM((CHUNK, d), F32),
            pltpu.VMEM((CHUNK, d), F32),
            pltpu.VMEM((zr, d), F32),
            pltpu.VMEM_SHARED((n_pad, d), F32),
            pltpu.SemaphoreType.DMA,
            pltpu.SemaphoreType.DMA,
        ],
    )
    def scatter_k(ue_hbm, ridx_hbm, out_hbm, ib0, ib1, rv0, rv1, zbuf,
                  acc_sh, sm0, sm1):
        cid = lax.axis_index("c")
        sid = lax.axis_index("s")
        wid = sid * 2 + cid
        B = ((ib0, rv0, sm0), (ib1, rv1, sm1))

        def issue(j, b):
            ib, rv, sm = B[b]
            c = j * NW + wid
            pltpu.sync_copy(ridx_hbm.at[pl.ds(c * CHUNK, CHUNK)], ib)
            pltpu.async_copy(ue_hbm.at[pl.ds(c * CHUNK, CHUNK)], rv, sm)

        def finish(j, b):
            ib, rv, sm = B[b]
            c = j * NW + wid
            pltpu.make_async_copy(
                ue_hbm.at[pl.ds(c * CHUNK, CHUNK)], rv, sm).wait()
            pltpu.sync_copy(rv, acc_sh.at[ib], add=True)

        issue(0, 0)

        def zb(i, carry):
            r = i // (d // 16)
            q = (i % (d // 16)) * 16
            zbuf[r, pl.ds(q, 16)] = jnp.zeros((16,), F32)
            return carry

        lax.fori_loop(0, zr * (d // 16), zb, 0)
        base = sid * rows_per_tile
        for t in range(n_zcopy):
            pltpu.sync_copy(zbuf, acc_sh.at[pl.ds(base + t * zr, zr)])
        plsc.subcore_barrier()

        def body(i, carry):
            j0 = 2 * i
            issue(j0 + 1, 1)
            finish(j0, 0)
            issue(j0 + 2, 0)
            finish(j0 + 1, 1)
            return carry

        lax.fori_loop(0, n_full // 2 - 1, body, 0)
        issue(n_full - 1, 1)
        finish(n_full - 2, 0)
        if extra:
            @pl.when(wid < extra)
            def _():
                issue(n_full, 0)
        finish(n_full - 1, 1)
        if extra:
            @pl.when(wid < extra)
            def _():
                finish(n_full, 0)
        plsc.subcore_barrier()
        for t in range(n_zcopy):
            sl = pl.ds(base + t * zr, zr)
            pltpu.sync_copy(acc_sh.at[sl], out_hbm.at[cid, sl])

    return scatter_k


def kernel(node_features, edge_features, senders, receivers,
           We1, be1, We2, be2, ln_e_scale, ln_e_bias,
           Wn1, bn1, Wn2, bn2, ln_n_scale, ln_n_bias):
    n, d = node_features.shape
    e = edge_features.shape[0]
    n_chunks = e // CHUNK

    sidx = senders.astype(jnp.int32)
    ridx = receivers.astype(jnp.int32)

    n_chunks_pad = ((n_chunks + 2 * NW - 1) // (2 * NW)) * 2 * NW
    e_pad = n_chunks_pad * CHUNK
    sidx_p = jnp.pad(sidx, (0, e_pad - e))
    ridx_p = jnp.pad(ridx, (0, e_pad - e))

    ws, wr, we = We1[:d], We1[d:2 * d], We1[2 * d:]
    ps, pr = _tc_proj(node_features, ws, wr)
    g2 = _make_gather(n, n_chunks, n_chunks_pad, d)(ps, pr, sidx_p, ridx_p)
    ue, ne = _tc_edge(edge_features, g2, we,
                      be1.reshape(1, d), We2, be2.reshape(1, d),
                      ln_e_scale.reshape(1, d), ln_e_bias.reshape(1, d))
    agg2 = _make_scatter(n, n_chunks, d)(ue, ridx)
    new_nodes = _tc_node(node_features, agg2[0, :n], agg2[1, :n],
                         Wn1[:d], Wn1[d:], bn1.reshape(1, d),
                         Wn2, bn2.reshape(1, d),
                         ln_n_scale.reshape(1, d), ln_n_bias.reshape(1, d))
    return (new_nodes, ne)


# speedup vs baseline: 1.1699x; 1.1071x over previous
import functools

import jax
import jax.numpy as jnp
from jax import lax
from jax.experimental import pallas as pl
from jax.experimental.pallas import tpu as pltpu
from jax.experimental.pallas import tpu_sc as plsc

F32 = jnp.float32
CHUNK = 128
NW = 32


def _proj_body(nf_ref, ws_ref, wr_ref, ps_ref, pr_ref):
    nf = nf_ref[...]
    ps_ref[...] = jnp.dot(nf, ws_ref[...], preferred_element_type=F32)
    pr_ref[...] = jnp.dot(nf, wr_ref[...], preferred_element_type=F32)


def _edge_body(ef_ref, gs_ref, gr_ref, we_ref, be1_ref, we2_ref, be2_ref,
               sc_ref, bi_ref, ue_ref, ne_ref):
    ef = ef_ref[...]
    x = (gs_ref[0] + gr_ref[0] + be1_ref[...]
         + jnp.dot(ef, we_ref[...], preferred_element_type=F32))
    h = jnp.maximum(x, 0.0)
    y = jnp.dot(h, we2_ref[...], preferred_element_type=F32) + be2_ref[...]
    mean = jnp.mean(y, axis=-1, keepdims=True)
    var = jnp.mean((y - mean) ** 2, axis=-1, keepdims=True)
    ue = (y - mean) / jnp.sqrt(var + 1e-5) * sc_ref[...] + bi_ref[...]
    ue_ref[...] = ue
    ne_ref[...] = ue + ef


def _node_body(nf_ref, p0_ref, p1_ref, wa_ref, wb_ref, bn1_ref, wn2_ref,
               bn2_ref, sc_ref, bi_ref, out_ref):
    nf = nf_ref[...]
    agg = p0_ref[...] + p1_ref[...]
    x = (jnp.dot(nf, wa_ref[...], preferred_element_type=F32)
         + jnp.dot(agg, wb_ref[...], preferred_element_type=F32)
         + bn1_ref[...])
    h = jnp.maximum(x, 0.0)
    y = jnp.dot(h, wn2_ref[...], preferred_element_type=F32) + bn2_ref[...]
    mean = jnp.mean(y, axis=-1, keepdims=True)
    var = jnp.mean((y - mean) ** 2, axis=-1, keepdims=True)
    out_ref[...] = ((y - mean) / jnp.sqrt(var + 1e-5) * sc_ref[...]
                    + bi_ref[...] + nf)


def _tc_proj(nf, ws, wr):
    n, d = nf.shape
    bn = 2000
    return pl.pallas_call(
        _proj_body,
        grid=(n // bn,),
        in_specs=[
            pl.BlockSpec((bn, d), lambda i: (i, 0)),
            pl.BlockSpec((d, d), lambda i: (0, 0)),
            pl.BlockSpec((d, d), lambda i: (0, 0)),
        ],
        out_specs=(pl.BlockSpec((bn, d), lambda i: (i, 0)),
                   pl.BlockSpec((bn, d), lambda i: (i, 0))),
        out_shape=(jax.ShapeDtypeStruct((n, d), F32),
                   jax.ShapeDtypeStruct((n, d), F32)),
    )(nf, ws, wr)


def _tc_edge(ef, g2, we, be1, we2, be2, sc, bi):
    e, d = ef.shape
    be = 2000
    row = lambda i: (i, 0)
    cst = lambda i: (0, 0)
    return pl.pallas_call(
        _edge_body,
        grid=(e // be,),
        in_specs=[
            pl.BlockSpec((be, d), row),
            pl.BlockSpec((1, be, d), lambda i: (0, i, 0)),
            pl.BlockSpec((1, be, d), lambda i: (1, i, 0)),
            pl.BlockSpec((d, d), cst),
            pl.BlockSpec((1, d), cst),
            pl.BlockSpec((d, d), cst),
            pl.BlockSpec((1, d), cst),
            pl.BlockSpec((1, d), cst),
            pl.BlockSpec((1, d), cst),
        ],
        out_specs=(pl.BlockSpec((be, d), row), pl.BlockSpec((be, d), row)),
        out_shape=(jax.ShapeDtypeStruct((e, d), F32),
                   jax.ShapeDtypeStruct((e, d), F32)),
    )(ef, g2, g2, we, be1, we2, be2, sc, bi)


def _tc_node(nf, p0, p1, wa, wb, bn1, wn2, bn2, sc, bi):
    n, d = nf.shape
    bn = 2000
    row = lambda i: (i, 0)
    cst = lambda i: (0, 0)
    return pl.pallas_call(
        _node_body,
        grid=(n // bn,),
        in_specs=[
            pl.BlockSpec((bn, d), row),
            pl.BlockSpec((bn, d), row),
            pl.BlockSpec((bn, d), row),
            pl.BlockSpec((d, d), cst),
            pl.BlockSpec((d, d), cst),
            pl.BlockSpec((1, d), cst),
            pl.BlockSpec((d, d), cst),
            pl.BlockSpec((1, d), cst),
            pl.BlockSpec((1, d), cst),
            pl.BlockSpec((1, d), cst),
        ],
        out_specs=pl.BlockSpec((bn, d), row),
        out_shape=jax.ShapeDtypeStruct((n, d), F32),
    )(nf, p0, p1, wa, wb, bn1, wn2, bn2, sc, bi)


@functools.lru_cache(maxsize=None)
def _make_gather(n_nodes, n_chunks, d):
    mesh = plsc.VectorSubcoreMesh(core_axis_name="c", subcore_axis_name="s")
    e_pad = n_chunks * CHUNK
    NS = 16
    n_full = n_chunks // NS
    assert n_full >= 4 and n_full % 2 == 0
    stripe = ((n_nodes // NS) // 8 + 1) * 8
    last = n_nodes - 15 * stripe
    assert 0 < last <= stripe

    @functools.partial(
        pl.kernel,
        out_type=jax.ShapeDtypeStruct((2, e_pad, d), F32),
        mesh=mesh,
        scratch_types=[
            pltpu.VMEM((CHUNK,), jnp.int32),
            pltpu.VMEM((CHUNK,), jnp.int32),
            pltpu.VMEM((CHUNK, d), F32),
            pltpu.VMEM((CHUNK, d), F32),
            pltpu.VMEM_SHARED((n_nodes, d), F32),
            pltpu.SemaphoreType.DMA,
            pltpu.SemaphoreType.DMA,
            pltpu.SemaphoreType.DMA,
            pltpu.SemaphoreType.DMA,
            pltpu.SemaphoreType.DMA,
            pltpu.SemaphoreType.DMA,
        ],
    )
    def gather_k(ps_hbm, pr_hbm, sidx_hbm, ridx_hbm, out_hbm,
                 ib0, ib1, rv0, rv1, tbl_sh, is0, is1, gs0, gs1, ws0, ws1):
        cid = lax.axis_index("c")
        sid = lax.axis_index("s")
        B = ((ib0, rv0, is0, gs0, ws0), (ib1, rv1, is1, gs1, ws1))

        @pl.when(sid < 15)
        def _():
            sl = pl.ds(sid * stripe, stripe)

            @pl.when(cid == 0)
            def _():
                pltpu.sync_copy(ps_hbm.at[sl], tbl_sh.at[sl])

            @pl.when(cid == 1)
            def _():
                pltpu.sync_copy(pr_hbm.at[sl], tbl_sh.at[sl])

        @pl.when(sid == 15)
        def _():
            sl = pl.ds(15 * stripe, last)

            @pl.when(cid == 0)
            def _():
                pltpu.sync_copy(ps_hbm.at[sl], tbl_sh.at[sl])

            @pl.when(cid == 1)
            def _():
                pltpu.sync_copy(pr_hbm.at[sl], tbl_sh.at[sl])

        plsc.subcore_barrier()

        def load_idx(j, b):
            ib, _, isem, _, _ = B[b]
            c = j * NS + sid

            @pl.when(cid == 0)
            def _():
                pltpu.async_copy(sidx_hbm.at[pl.ds(c * CHUNK, CHUNK)], ib,
                                 isem)

            @pl.when(cid == 1)
            def _():
                pltpu.async_copy(ridx_hbm.at[pl.ds(c * CHUNK, CHUNK)], ib,
                                 isem)

        def wait_idx(b):
            ib, _, isem, _, _ = B[b]
            pltpu.make_async_copy(
                sidx_hbm.at[pl.ds(0, CHUNK)], ib, isem).wait()

        def stage(j, b, first):
            ib, rv, isem, gsem, wsem = B[b]
            ib2, rv2, isem2, gsem2, wsem2 = B[1 - b]
            c = j * NS + sid
            pltpu.make_async_copy(tbl_sh.at[ib], rv, gsem).wait()
            load_idx(j + 2, b)
            wait_idx(1 - b)
            if not first:
                pltpu.make_async_copy(
                    rv2, out_hbm.at[cid, pl.ds(0, CHUNK)], wsem2).wait()
            pltpu.async_copy(tbl_sh.at[ib2], rv2, gsem2)
            pltpu.async_copy(rv, out_hbm.at[cid, pl.ds(c * CHUNK, CHUNK)],
                             wsem)

        load_idx(0, 0)
        wait_idx(0)
        pltpu.async_copy(tbl_sh.at[ib0], rv0, gs0)
        load_idx(1, 1)
        stage(0, 0, True)
        stage(1, 1, False)

        def body(i, carry):
            stage(2 * i, 0, False)
            stage(2 * i + 1, 1, False)
            return carry

        lax.fori_loop(1, n_full // 2, body, 0)
        pltpu.make_async_copy(tbl_sh.at[ib0], rv0, gs0).wait()
        pltpu.make_async_copy(rv1, out_hbm.at[cid, pl.ds(0, CHUNK)],
                              ws1).wait()
        pltpu.make_async_copy(sidx_hbm.at[pl.ds(0, CHUNK)], ib1, is1).wait()

    return gather_k


@functools.lru_cache(maxsize=None)
def _make_scatter(n_nodes, n_chunks, d):
    mesh = plsc.VectorSubcoreMesh(core_axis_name="c", subcore_axis_name="s")
    n_pad = ((n_nodes + 16 * 128 - 1) // (16 * 128)) * 16 * 128
    rows_per_tile = n_pad // 16
    zr = 64
    n_zcopy = rows_per_tile // zr

    n_full = n_chunks // NW
    extra = n_chunks % NW
    assert n_full >= 4 and n_full % 2 == 0

    @functools.partial(
        pl.kernel,
        out_type=jax.ShapeDtypeStruct((2, n_pad, d), F32),
        mesh=mesh,
        scratch_types=[
            pltpu.VMEM((CHUNK,), jnp.int32),
            pltpu.VMEM((CHUNK,), jnp.int32),
            pltpu.VMEM((CHUNK, d), F32),
            pltpu.VMEM((CHUNK, d), F32),
            pltpu.VMEM((zr, d), F32),
            pltpu.VMEM_SHARED((n_pad, d), F32),
            pltpu.SemaphoreType.DMA,
            pltpu.SemaphoreType.DMA,
        ],
    )
    def scatter_k(ue_hbm, ridx_hbm, out_hbm, ib0, ib1, rv0, rv1, zbuf,
                  acc_sh, sm0, sm1):
        cid = lax.axis_index("c")
        sid = lax.axis_index("s")
        wid = sid * 2 + cid
        B = ((ib0, rv0, sm0), (ib1, rv1, sm1))

        def issue(j, b):
            ib, rv, sm = B[b]
            c = j * NW + wid
            pltpu.sync_copy(ridx_hbm.at[pl.ds(c * CHUNK, CHUNK)], ib)
            pltpu.async_copy(ue_hbm.at[pl.ds(c * CHUNK, CHUNK)], rv, sm)

        def finish(j, b):
            ib, rv, sm = B[b]
            c = j * NW + wid
            pltpu.make_async_copy(
                ue_hbm.at[pl.ds(c * CHUNK, CHUNK)], rv, sm).wait()
            pltpu.sync_copy(rv, acc_sh.at[ib], add=True)

        issue(0, 0)

        def zb(i, carry):
            r = i // (d // 16)
            q = (i % (d // 16)) * 16
            zbuf[r, pl.ds(q, 16)] = jnp.zeros((16,), F32)
            return carry

        lax.fori_loop(0, zr * (d // 16), zb, 0)
        base = sid * rows_per_tile
        for t in range(n_zcopy):
            pltpu.sync_copy(zbuf, acc_sh.at[pl.ds(base + t * zr, zr)])
        plsc.subcore_barrier()

        def body(i, carry):
            j0 = 2 * i
            issue(j0 + 1, 1)
            finish(j0, 0)
            issue(j0 + 2, 0)
            finish(j0 + 1, 1)
            return carry

        lax.fori_loop(0, n_full // 2 - 1, body, 0)
        issue(n_full - 1, 1)
        finish(n_full - 2, 0)
        if extra:
            @pl.when(wid < extra)
            def _():
                issue(n_full, 0)
        finish(n_full - 1, 1)
        if extra:
            @pl.when(wid < extra)
            def _():
                finish(n_full, 0)
        plsc.subcore_barrier()
        for t in range(n_zcopy):
            sl = pl.ds(base + t * zr, zr)
            pltpu.sync_copy(acc_sh.at[sl], out_hbm.at[cid, sl])

    return scatter_k


def kernel(node_features, edge_features, senders, receivers,
           We1, be1, We2, be2, ln_e_scale, ln_e_bias,
           Wn1, bn1, Wn2, bn2, ln_n_scale, ln_n_bias):
    n, d = node_features.shape
    e = edge_features.shape[0]
    n_chunks = e // CHUNK

    sidx = senders.astype(jnp.int32)
    ridx = receivers.astype(jnp.int32)

    ns2 = 2 * 16
    n_chunks_g = ((n_chunks + ns2 - 1) // ns2) * ns2
    e_pad = (n_chunks_g + ns2) * CHUNK
    sidx_p = jnp.pad(sidx, (0, e_pad - e))
    ridx_p = jnp.pad(ridx, (0, e_pad - e))

    ws, wr, we = We1[:d], We1[d:2 * d], We1[2 * d:]
    ps, pr = _tc_proj(node_features, ws, wr)
    g2 = _make_gather(n, n_chunks_g, d)(ps, pr, sidx_p, ridx_p)
    ue, ne = _tc_edge(edge_features, g2, we,
                      be1.reshape(1, d), We2, be2.reshape(1, d),
                      ln_e_scale.reshape(1, d), ln_e_bias.reshape(1, d))
    agg2 = _make_scatter(n, n_chunks, d)(ue, ridx)
    new_nodes = _tc_node(node_features, agg2[0, :n], agg2[1, :n],
                         Wn1[:d], Wn1[d:], bn1.reshape(1, d),
                         Wn2, bn2.reshape(1, d),
                         ln_n_scale.reshape(1, d), ln_n_bias.reshape(1, d))
    return (new_nodes, ne)


# speedup vs baseline: 1.1935x; 1.0202x over previous
import functools

import jax
import jax.numpy as jnp
from jax import lax
from jax.experimental import pallas as pl
from jax.experimental.pallas import tpu as pltpu
from jax.experimental.pallas import tpu_sc as plsc

F32 = jnp.float32
CHUNK = 128
NW = 32


def _proj_body(nf_ref, ws_ref, wr_ref, ps_ref, pr_ref):
    nf = nf_ref[...]
    ps_ref[...] = jnp.dot(nf, ws_ref[...], preferred_element_type=F32)
    pr_ref[...] = jnp.dot(nf, wr_ref[...], preferred_element_type=F32)


def _edge_body(ef_ref, gs_ref, gr_ref, we_ref, be1_ref, we2_ref, be2_ref,
               sc_ref, bi_ref, ue_ref, ne_ref):
    ef = ef_ref[...]
    x = (gs_ref[0] + gr_ref[0] + be1_ref[...]
         + jnp.dot(ef, we_ref[...], preferred_element_type=F32))
    h = jnp.maximum(x, 0.0)
    y = jnp.dot(h, we2_ref[...], preferred_element_type=F32) + be2_ref[...]
    mean = jnp.mean(y, axis=-1, keepdims=True)
    var = jnp.mean((y - mean) ** 2, axis=-1, keepdims=True)
    ue = (y - mean) / jnp.sqrt(var + 1e-5) * sc_ref[...] + bi_ref[...]
    ue_ref[...] = ue
    ne_ref[...] = ue + ef


def _node_body(nf_ref, p0_ref, p1_ref, wa_ref, wb_ref, bn1_ref, wn2_ref,
               bn2_ref, sc_ref, bi_ref, out_ref):
    nf = nf_ref[...]
    agg = p0_ref[...] + p1_ref[...]
    x = (jnp.dot(nf, wa_ref[...], preferred_element_type=F32)
         + jnp.dot(agg, wb_ref[...], preferred_element_type=F32)
         + bn1_ref[...])
    h = jnp.maximum(x, 0.0)
    y = jnp.dot(h, wn2_ref[...], preferred_element_type=F32) + bn2_ref[...]
    mean = jnp.mean(y, axis=-1, keepdims=True)
    var = jnp.mean((y - mean) ** 2, axis=-1, keepdims=True)
    out_ref[...] = ((y - mean) / jnp.sqrt(var + 1e-5) * sc_ref[...]
                    + bi_ref[...] + nf)


def _tc_proj(nf, ws, wr):
    n, d = nf.shape
    bn = 2000
    return pl.pallas_call(
        _proj_body,
        grid=(n // bn,),
        in_specs=[
            pl.BlockSpec((bn, d), lambda i: (i, 0)),
            pl.BlockSpec((d, d), lambda i: (0, 0)),
            pl.BlockSpec((d, d), lambda i: (0, 0)),
        ],
        out_specs=(pl.BlockSpec((bn, d), lambda i: (i, 0)),
                   pl.BlockSpec((bn, d), lambda i: (i, 0))),
        out_shape=(jax.ShapeDtypeStruct((n, d), F32),
                   jax.ShapeDtypeStruct((n, d), F32)),
    )(nf, ws, wr)


def _tc_edge(ef, g2, we, be1, we2, be2, sc, bi):
    e, d = ef.shape
    be = 2000
    row = lambda i: (i, 0)
    cst = lambda i: (0, 0)
    return pl.pallas_call(
        _edge_body,
        grid=(e // be,),
        in_specs=[
            pl.BlockSpec((be, d), row),
            pl.BlockSpec((1, be, d), lambda i: (0, i, 0)),
            pl.BlockSpec((1, be, d), lambda i: (1, i, 0)),
            pl.BlockSpec((d, d), cst),
            pl.BlockSpec((1, d), cst),
            pl.BlockSpec((d, d), cst),
            pl.BlockSpec((1, d), cst),
            pl.BlockSpec((1, d), cst),
            pl.BlockSpec((1, d), cst),
        ],
        out_specs=(pl.BlockSpec((be, d), row), pl.BlockSpec((be, d), row)),
        out_shape=(jax.ShapeDtypeStruct((e, d), F32),
                   jax.ShapeDtypeStruct((e, d), F32)),
    )(ef, g2, g2, we, be1, we2, be2, sc, bi)


def _tc_node(nf, p0, p1, wa, wb, bn1, wn2, bn2, sc, bi):
    n, d = nf.shape
    bn = 2000
    row = lambda i: (i, 0)
    cst = lambda i: (0, 0)
    return pl.pallas_call(
        _node_body,
        grid=(n // bn,),
        in_specs=[
            pl.BlockSpec((bn, d), row),
            pl.BlockSpec((bn, d), row),
            pl.BlockSpec((bn, d), row),
            pl.BlockSpec((d, d), cst),
            pl.BlockSpec((d, d), cst),
            pl.BlockSpec((1, d), cst),
            pl.BlockSpec((d, d), cst),
            pl.BlockSpec((1, d), cst),
            pl.BlockSpec((1, d), cst),
            pl.BlockSpec((1, d), cst),
        ],
        out_specs=pl.BlockSpec((bn, d), row),
        out_shape=jax.ShapeDtypeStruct((n, d), F32),
    )(nf, p0, p1, wa, wb, bn1, wn2, bn2, sc, bi)


@functools.lru_cache(maxsize=None)
def _make_gather(n_nodes, n_chunks, d):
    mesh = plsc.VectorSubcoreMesh(core_axis_name="c", subcore_axis_name="s")
    e_pad = n_chunks * CHUNK
    NS = 16
    n_full = n_chunks // NS
    assert n_full >= 4 and n_full % 2 == 0
    stripe = ((n_nodes // NS) // 8 + 1) * 8
    last = n_nodes - 15 * stripe
    assert 0 < last <= stripe

    @functools.partial(
        pl.kernel,
        out_type=jax.ShapeDtypeStruct((2, e_pad, d), F32),
        mesh=mesh,
        scratch_types=[
            pltpu.VMEM((CHUNK,), jnp.int32),
            pltpu.VMEM((CHUNK,), jnp.int32),
            pltpu.VMEM((CHUNK, d), F32),
            pltpu.VMEM((CHUNK, d), F32),
            pltpu.VMEM_SHARED((n_nodes, d), F32),
            pltpu.SemaphoreType.DMA,
            pltpu.SemaphoreType.DMA,
            pltpu.SemaphoreType.DMA,
            pltpu.SemaphoreType.DMA,
            pltpu.SemaphoreType.DMA,
            pltpu.SemaphoreType.DMA,
        ],
    )
    def gather_k(ps_hbm, pr_hbm, sidx_hbm, ridx_hbm, out_hbm,
                 ib0, ib1, rv0, rv1, tbl_sh, is0, is1, gs0, gs1, ws0, ws1):
        cid = lax.axis_index("c")
        sid = lax.axis_index("s")
        B = ((ib0, rv0, is0, gs0, ws0), (ib1, rv1, is1, gs1, ws1))

        @pl.when(sid < 15)
        def _():
            sl = pl.ds(sid * stripe, stripe)

            @pl.when(cid == 0)
            def _():
                pltpu.sync_copy(ps_hbm.at[sl], tbl_sh.at[sl])

            @pl.when(cid == 1)
            def _():
                pltpu.sync_copy(pr_hbm.at[sl], tbl_sh.at[sl])

        @pl.when(sid == 15)
        def _():
            sl = pl.ds(15 * stripe, last)

            @pl.when(cid == 0)
            def _():
                pltpu.sync_copy(ps_hbm.at[sl], tbl_sh.at[sl])

            @pl.when(cid == 1)
            def _():
                pltpu.sync_copy(pr_hbm.at[sl], tbl_sh.at[sl])

        plsc.subcore_barrier()

        def load_idx(j, b):
            ib, _, isem, _, _ = B[b]
            c = j * NS + sid

            @pl.when(cid == 0)
            def _():
                pltpu.async_copy(sidx_hbm.at[pl.ds(c * CHUNK, CHUNK)], ib,
                                 isem)

            @pl.when(cid == 1)
            def _():
                pltpu.async_copy(ridx_hbm.at[pl.ds(c * CHUNK, CHUNK)], ib,
                                 isem)

        def wait_idx(b):
            ib, _, isem, _, _ = B[b]
            pltpu.make_async_copy(
                sidx_hbm.at[pl.ds(0, CHUNK)], ib, isem).wait()

        def stage(j, b, first):
            ib, rv, isem, gsem, wsem = B[b]
            ib2, rv2, isem2, gsem2, wsem2 = B[1 - b]
            c = j * NS + sid
            pltpu.make_async_copy(tbl_sh.at[ib], rv, gsem).wait()
            load_idx(j + 2, b)
            wait_idx(1 - b)
            if not first:
                pltpu.make_async_copy(
                    rv2, out_hbm.at[cid, pl.ds(0, CHUNK)], wsem2).wait()
            pltpu.async_copy(tbl_sh.at[ib2], rv2, gsem2)
            pltpu.async_copy(rv, out_hbm.at[cid, pl.ds(c * CHUNK, CHUNK)],
                             wsem)

        load_idx(0, 0)
        wait_idx(0)
        pltpu.async_copy(tbl_sh.at[ib0], rv0, gs0)
        load_idx(1, 1)
        stage(0, 0, True)
        stage(1, 1, False)

        def body(i, carry):
            stage(2 * i, 0, False)
            stage(2 * i + 1, 1, False)
            return carry

        lax.fori_loop(1, n_full // 2, body, 0)
        pltpu.make_async_copy(tbl_sh.at[ib0], rv0, gs0).wait()
        pltpu.make_async_copy(rv1, out_hbm.at[cid, pl.ds(0, CHUNK)],
                              ws1).wait()
        pltpu.make_async_copy(sidx_hbm.at[pl.ds(0, CHUNK)], ib1, is1).wait()

    return gather_k


@functools.lru_cache(maxsize=None)
def _make_scatter(n_nodes, n_chunks, d):
    mesh = plsc.VectorSubcoreMesh(core_axis_name="c", subcore_axis_name="s")
    n_pad = ((n_nodes + 127) // 128) * 128
    rows_per_tile = n_pad // 16
    nz_full = rows_per_tile // CHUNK
    nz_rem = rows_per_tile % CHUNK

    n_full = n_chunks // NW
    extra = n_chunks % NW
    assert n_full >= 6 and n_full % 3 == 0

    @functools.partial(
        pl.kernel,
        out_type=jax.ShapeDtypeStruct((2, n_pad, d), F32),
        mesh=mesh,
        scratch_types=[
            pltpu.VMEM((CHUNK,), jnp.int32),
            pltpu.VMEM((CHUNK,), jnp.int32),
            pltpu.VMEM((CHUNK,), jnp.int32),
            pltpu.VMEM((CHUNK, d), F32),
            pltpu.VMEM((CHUNK, d), F32),
            pltpu.VMEM((CHUNK, d), F32),
            pltpu.VMEM_SHARED((n_pad, d), F32),
            pltpu.SemaphoreType.DMA,
            pltpu.SemaphoreType.DMA,
            pltpu.SemaphoreType.DMA,
            pltpu.SemaphoreType.DMA,
            pltpu.SemaphoreType.DMA,
            pltpu.SemaphoreType.DMA,
            pltpu.SemaphoreType.DMA,
            pltpu.SemaphoreType.DMA,
            pltpu.SemaphoreType.DMA,
        ],
    )
    def scatter_k(ue_hbm, ridx_hbm, out_hbm, ib0, ib1, ib2, rv0, rv1, rv2,
                  acc_sh, is0, is1, is2, rs0, rs1, rs2, ss0, ss1, ss2):
        cid = lax.axis_index("c")
        sid = lax.axis_index("s")
        wid = sid * 2 + cid
        B = ((ib0, rv0, is0, rs0, ss0),
             (ib1, rv1, is1, rs1, ss1),
             (ib2, rv2, is2, rs2, ss2))

        def issue(j, b):
            ib, rv, isem, rsem, _ = B[b]
            c = j * NW + wid

            @pl.when(c < n_chunks)
            def _():
                pltpu.async_copy(ridx_hbm.at[pl.ds(c * CHUNK, CHUNK)], ib,
                                 isem)
                pltpu.async_copy(ue_hbm.at[pl.ds(c * CHUNK, CHUNK)], rv,
                                 rsem)

        def drain_add(b):
            ib, rv, _, _, ssem = B[b]
            pltpu.make_async_copy(rv, acc_sh.at[ib], ssem).wait()

        def stage(j, b, first):
            ib, rv, isem, rsem, ssem = B[b]
            b2 = (b + 2) % 3
            pltpu.make_async_copy(
                ridx_hbm.at[pl.ds(0, CHUNK)], ib, isem).wait()
            pltpu.make_async_copy(
                ue_hbm.at[pl.ds(0, CHUNK)], rv, rsem).wait()
            pltpu.async_copy(rv, acc_sh.at[ib], ssem, add=True)
            if not first:
                drain_add(b2)
            issue(j + 2, b2)

        def zb(i, carry):
            r = i // (d // 16)
            q = (i % (d // 16)) * 16
            rv0[r, pl.ds(q, 16)] = jnp.zeros((16,), F32)
            return carry

        lax.fori_loop(0, CHUNK * (d // 16), zb, 0)
        base = sid * rows_per_tile
        for t in range(nz_full):
            pltpu.sync_copy(rv0, acc_sh.at[pl.ds(base + t * CHUNK, CHUNK)])
        if nz_rem:
            pltpu.sync_copy(
                rv0.at[pl.ds(0, nz_rem)],
                acc_sh.at[pl.ds(base + nz_full * CHUNK, nz_rem)])
        plsc.subcore_barrier()

        issue(0, 0)
        issue(1, 1)
        stage(0, 0, True)
        stage(1, 1, False)
        stage(2, 2, False)

        def body(i, carry):
            j0 = 3 * i
            stage(j0, 0, False)
            stage(j0 + 1, 1, False)
            stage(j0 + 2, 2, False)
            return carry

        lax.fori_loop(1, n_full // 3, body, 0)
        if extra:
            @pl.when(wid < extra)
            def _():
                stage(n_full, n_full % 3, True)
        drain_add((n_full - 1) % 3)
        if extra:
            @pl.when(wid < extra)
            def _():
                drain_add(n_full % 3)
        plsc.subcore_barrier()
        for t in range(nz_full):
            sl = pl.ds(base + t * CHUNK, CHUNK)
            pltpu.sync_copy(acc_sh.at[sl], out_hbm.at[cid, sl])
        if nz_rem:
            sl = pl.ds(base + nz_full * CHUNK, nz_rem)
            pltpu.sync_copy(acc_sh.at[sl], out_hbm.at[cid, sl])

    return scatter_k


def kernel(node_features, edge_features, senders, receivers,
           We1, be1, We2, be2, ln_e_scale, ln_e_bias,
           Wn1, bn1, Wn2, bn2, ln_n_scale, ln_n_bias):
    n, d = node_features.shape
    e = edge_features.shape[0]
    n_chunks = e // CHUNK

    sidx = senders.astype(jnp.int32)
    ridx = receivers.astype(jnp.int32)

    ns2 = 2 * 16
    n_chunks_g = ((n_chunks + ns2 - 1) // ns2) * ns2
    e_pad = (n_chunks_g + ns2) * CHUNK
    sidx_p = jnp.pad(sidx, (0, e_pad - e))
    ridx_p = jnp.pad(ridx, (0, e_pad - e))

    ws, wr, we = We1[:d], We1[d:2 * d], We1[2 * d:]
    ps, pr = _tc_proj(node_features, ws, wr)
    g2 = _make_gather(n, n_chunks_g, d)(ps, pr, sidx_p, ridx_p)
    ue, ne = _tc_edge(edge_features, g2, we,
                      be1.reshape(1, d), We2, be2.reshape(1, d),
                      ln_e_scale.reshape(1, d), ln_e_bias.reshape(1, d))
    agg2 = _make_scatter(n, n_chunks, d)(ue, ridx)
    new_nodes = _tc_node(node_features, agg2[0, :n], agg2[1, :n],
                         Wn1[:d], Wn1[d:], bn1.reshape(1, d),
                         Wn2, bn2.reshape(1, d),
                         ln_n_scale.reshape(1, d), ln_n_bias.reshape(1, d))
    return (new_nodes, ne)
